# baseline (speedup 1.0000x reference)
import jax
import jax.numpy as jnp
from jax.experimental import pallas as pl


_B = 256


def _body(x_ref, o_ref):
    i = pl.program_id(1)
    j = pl.program_id(2)

    @pl.when(j < i)
    def _():
        o_ref[...] = x_ref[...]

    @pl.when(j == i)
    def _():
        r = jax.lax.broadcasted_iota(jnp.int32, (1, _B, _B), 1)
        c = jax.lax.broadcasted_iota(jnp.int32, (1, _B, _B), 2)
        o_ref[...] = jnp.where(c > r, jnp.float32(1.0), x_ref[...])

    @pl.when(j > i)
    def _():
        o_ref[...] = jnp.ones((1, _B, _B), jnp.float32)


def kernel(x):
    batch, s, _ = x.shape
    n = s // _B
    return pl.pallas_call(
        _body,
        grid=(batch, n, n),
        in_specs=[
            pl.BlockSpec((1, _B, _B), lambda b, i, j: (b, i, jnp.minimum(i, j)))
        ],
        out_specs=pl.BlockSpec((1, _B, _B), lambda b, i, j: (b, i, j)),
        out_shape=jax.ShapeDtypeStruct(x.shape, x.dtype),
    )(x)


# speedup vs baseline: 3.6305x; 3.6305x over previous
import jax
import jax.numpy as jnp
from jax.experimental import pallas as pl


_B = 512


def _body(x_ref, o_ref):
    i = pl.program_id(0)
    j = pl.program_id(1)

    @pl.when(j < i)
    def _():
        o_ref[...] = x_ref[...]

    @pl.when(j == i)
    def _():
        r = jax.lax.broadcasted_iota(jnp.int32, (1, _B, _B), 1)
        c = jax.lax.broadcasted_iota(jnp.int32, (1, _B, _B), 2)
        o_ref[...] = jnp.where(c > r, jnp.float32(1.0), x_ref[...])

    @pl.when(j > i)
    def _():
        o_ref[...] = jnp.ones(o_ref.shape, jnp.float32)


def kernel(x):
    batch, s, _ = x.shape
    n = s // _B
    return pl.pallas_call(
        _body,
        grid=(n, n),
        in_specs=[
            pl.BlockSpec((batch, _B, _B), lambda i, j: (0, i, jnp.minimum(i, j)))
        ],
        out_specs=pl.BlockSpec((batch, _B, _B), lambda i, j: (0, i, j)),
        out_shape=jax.ShapeDtypeStruct(x.shape, x.dtype),
    )(x)
